# Initial kernel scaffold; baseline (speedup 1.0000x reference)
import jax
import jax.numpy as jnp
from jax.experimental import pallas as pl


def kernel(pred, target):
    raise NotImplementedError("write your pallas kernel here")


# speedup vs baseline: 1.9769x; 1.9769x over previous
import jax
import jax.numpy as jnp
from jax.experimental import pallas as pl
from jax.experimental.pallas import tpu as pltpu

ROWS = 1024
COLS = 100000
BLOCK_R = 512
BLOCK_C = 2048
NJ = (COLS + BLOCK_C - 1) // BLOCK_C


def _body(tgt_ref, x_ref, o_ref):
    j = pl.program_id(1)
    x = x_ref[...]
    logsig = -jnp.log1p(jnp.exp(-x))
    cols = j * BLOCK_C + jax.lax.broadcasted_iota(jnp.int32, x.shape, 1)
    t = tgt_ref[...]
    valid = (cols != t[:, None]) & (cols < COLS)
    part = jnp.sum(jnp.where(valid, logsig, 0.0), axis=1)

    @pl.when(j == 0)
    def _():
        o_ref[...] = part

    @pl.when(j > 0)
    def _():
        o_ref[...] += part

    @pl.when(j == NJ - 1)
    def _():
        o_ref[...] = o_ref[...] * (-1.0 / ROWS)


@jax.jit
def kernel(pred, target):
    target = target.astype(jnp.int32)
    return pl.pallas_call(
        _body,
        grid=(ROWS // BLOCK_R, NJ),
        in_specs=[
            pl.BlockSpec((BLOCK_R,), lambda i, j: (i,)),
            pl.BlockSpec((BLOCK_R, BLOCK_C), lambda i, j: (i, j)),
        ],
        out_specs=pl.BlockSpec((BLOCK_R,), lambda i, j: (i,)),
        out_shape=jax.ShapeDtypeStruct((ROWS,), jnp.float32),
        compiler_params=pltpu.CompilerParams(
            dimension_semantics=("parallel", "arbitrary"),
        ),
    )(target, pred)
